# baseline (speedup 1.0000x reference)
import functools

import jax
import jax.numpy as jnp
from jax.experimental import pallas as pl

_NUM_LAYERS = 7
_HALF = 4
_N = 128
_G = 8

_dot = functools.partial(jnp.dot, preferred_element_type=jnp.float32)


def _body(nodes_ref, adj_ref, *refs):
    out_ref = refs[-1]
    wrefs = [r[...] for r in refs[:-1]]

    idx = 0
    layers = []
    for i in range(_NUM_LAYERS):
        n = 3 if i < _HALF else 5
        layers.append(tuple(wrefs[idx:idx + n]))
        idx += n
    Wf1a, Wf1b, Wf1c, Wf1d, bf1, Wf2, bf2 = wrefs[idx:]

    adjns = []
    for g in range(_G):
        a = adj_ref[g]
        deg = jnp.maximum(jnp.sum(a, axis=1, keepdims=True), 1e-6)
        adjns.append(a / deg)

    h = nodes_ref[...].reshape(_G * _N, nodes_ref.shape[2])
    outs = []
    for i in range(_NUM_LAYERS):
        if i < _HALF:
            Wt, Wb, b = layers[i]
            q = _dot(h, Wt)
            p = _dot(h, Wb)
        else:
            Wth, Wts, Wbh, Wbs, b = layers[i]
            s = outs[_NUM_LAYERS - 1 - i]
            q = _dot(h, Wth) + _dot(s, Wts)
            p = _dot(h, Wbh) + _dot(s, Wbs)
        nb = jnp.concatenate(
            [_dot(adjns[g], p[g * _N:(g + 1) * _N]) for g in range(_G)], axis=0)
        h = jax.nn.relu(q + nb + b)
        outs.append(h)

    mxs, mns, sms = [], [], []
    for g in range(_G):
        hg = h[g * _N:(g + 1) * _N]
        mxs.append(jnp.max(hg, axis=0, keepdims=True))
        mns.append(jnp.min(hg, axis=0, keepdims=True))
        sms.append(jnp.sum(hg, axis=0, keepdims=True))
    MX = jnp.concatenate(mxs, axis=0)
    MN = jnp.concatenate(mns, axis=0)
    SM = jnp.concatenate(sms, axis=0)
    AV = SM * (1.0 / _N)
    hid = jax.nn.relu(
        _dot(MX, Wf1a) + _dot(MN, Wf1b) + _dot(AV, Wf1c) + _dot(SM, Wf1d) + bf1)
    out_ref[...] = _dot(hid, Wf2) + bf2


def kernel(nodes, adj, W0, W1, W2, W3, W4, W5, W6,
           b0, b1, b2, b3, b4, b5, b6, Wf1, bf1, Wf2, bf2):
    B, N, F0 = nodes.shape
    Ws = (W0, W1, W2, W3, W4, W5, W6)
    bs = (b0, b1, b2, b3, b4, b5, b6)

    feat_dims = [F0] + [W.shape[1] for W in Ws]
    wargs = []
    for i in range(_NUM_LAYERS):
        d = Ws[i].shape[0] // 2
        Wt, Wb = Ws[i][:d], Ws[i][d:]
        if i < _HALF:
            wargs += [Wt, Wb]
        else:
            hd = feat_dims[i]
            wargs += [Wt[:hd], Wt[hd:], Wb[:hd], Wb[hd:]]
        wargs.append(bs[i].reshape(1, -1))
    k = Wf1.shape[0] // 4
    wargs += [Wf1[:k], Wf1[k:2 * k], Wf1[2 * k:3 * k], Wf1[3 * k:],
              bf1.reshape(1, -1), Wf2, bf2.reshape(1, -1)]

    grid = (B // _G,)
    const_spec = lambda a: pl.BlockSpec(a.shape, lambda i: (0,) * a.ndim)
    in_specs = [
        pl.BlockSpec((_G, N, F0), lambda i: (i, 0, 0)),
        pl.BlockSpec((_G, N, N), lambda i: (i, 0, 0)),
    ] + [const_spec(a) for a in wargs]

    out = pl.pallas_call(
        _body,
        grid=grid,
        in_specs=in_specs,
        out_specs=pl.BlockSpec((_G, Wf2.shape[1]), lambda i: (i, 0)),
        out_shape=jax.ShapeDtypeStruct((B, Wf2.shape[1]), jnp.float32),
    )(nodes, adj, *wargs)
    return out


# speedup vs baseline: 1.0601x; 1.0601x over previous
import functools

import jax
import jax.numpy as jnp
from jax.experimental import pallas as pl

_NUM_LAYERS = 7
_HALF = 4
_N = 128
_G = 8

_dot = functools.partial(jax.lax.dot_general,
                         preferred_element_type=jnp.float32)


def _nn(a, b):
    return _dot(a, b, (((1,), (0,)), ((), ())))


def _nt(a, b):
    return _dot(a, b, (((1,), (1,)), ((), ())))


def _body(nodes_ref, adj_ref, *refs):
    out_ref = refs[-1]
    wrefs = [r[...] for r in refs[:-1]]

    idx = 0
    layers = []
    for i in range(_NUM_LAYERS):
        n = 3 if i < _HALF else 5
        layers.append(tuple(wrefs[idx:idx + n]))
        idx += n
    Wf1a, Wf1b, Wf1c, Wf1d, bf1, Wf2, bf2 = wrefs[idx:]

    ones_row = jnp.ones((1, _N), jnp.float32)
    invdegs = []
    adjs = []
    for g in range(_G):
        a = adj_ref[g]
        adjs.append(a)
        deg = _nt(ones_row, a)
        invdegs.append(1.0 / jnp.maximum(deg, 1e-6))

    nodes = nodes_ref[...].reshape(_G * _N, nodes_ref.shape[2])

    hT = None
    outs = []
    for i in range(_NUM_LAYERS):
        if i < _HALF:
            WtT, WbT, bT = layers[i]
            if i == 0:
                qT = _nt(WtT, nodes)
                pT = _nt(WbT, nodes)
            else:
                qT = _nn(WtT, hT)
                pT = _nn(WbT, hT)
        else:
            WthT, WtsT, WbhT, WbsT, bT = layers[i]
            sT = outs[_NUM_LAYERS - 1 - i]
            qT = _nn(WthT, hT) + _nn(WtsT, sT)
            pT = _nn(WbhT, hT) + _nn(WbsT, sT)
        nbT = jnp.concatenate(
            [_nt(pT[:, g * _N:(g + 1) * _N], adjs[g]) * invdegs[g]
             for g in range(_G)], axis=1)
        hT = jax.nn.relu(qT + nbT + bT)
        outs.append(hT)

    mxs, mns, sms = [], [], []
    for g in range(_G):
        hg = hT[:, g * _N:(g + 1) * _N]
        mxs.append(jnp.max(hg, axis=1, keepdims=True))
        mns.append(jnp.min(hg, axis=1, keepdims=True))
        sms.append(jnp.sum(hg, axis=1, keepdims=True))
    MXT = jnp.concatenate(mxs, axis=1)
    MNT = jnp.concatenate(mns, axis=1)
    SMT = jnp.concatenate(sms, axis=1)
    AVT = SMT * (1.0 / _N)

    r = jax.lax.broadcasted_iota(jnp.int32, (_G, _G), 0)
    c = jax.lax.broadcasted_iota(jnp.int32, (_G, _G), 1)
    eye = (r == c).astype(jnp.float32)
    MX = _nt(eye, MXT)
    MN = _nt(eye, MNT)
    AV = _nt(eye, AVT)
    SM = _nt(eye, SMT)

    hid = jax.nn.relu(
        _nn(MX, Wf1a) + _nn(MN, Wf1b) + _nn(AV, Wf1c) + _nn(SM, Wf1d) + bf1)
    out_ref[...] = _nn(hid, Wf2) + bf2


def kernel(nodes, adj, W0, W1, W2, W3, W4, W5, W6,
           b0, b1, b2, b3, b4, b5, b6, Wf1, bf1, Wf2, bf2):
    B, N, F0 = nodes.shape
    Ws = (W0, W1, W2, W3, W4, W5, W6)
    bs = (b0, b1, b2, b3, b4, b5, b6)

    feat_dims = [F0] + [W.shape[1] for W in Ws]
    wargs = []
    for i in range(_NUM_LAYERS):
        d = Ws[i].shape[0] // 2
        Wt, Wb = Ws[i][:d], Ws[i][d:]
        if i < _HALF:
            wargs += [Wt.T, Wb.T]
        else:
            hd = feat_dims[i]
            wargs += [Wt[:hd].T, Wt[hd:].T, Wb[:hd].T, Wb[hd:].T]
        wargs.append(bs[i].reshape(-1, 1))
    k = Wf1.shape[0] // 4
    wargs += [Wf1[:k], Wf1[k:2 * k], Wf1[2 * k:3 * k], Wf1[3 * k:],
              bf1.reshape(1, -1), Wf2, bf2.reshape(1, -1)]

    grid = (B // _G,)
    const_spec = lambda a: pl.BlockSpec(a.shape, lambda i: (0,) * a.ndim)
    in_specs = [
        pl.BlockSpec((_G, N, F0), lambda i: (i, 0, 0)),
        pl.BlockSpec((_G, N, N), lambda i: (i, 0, 0)),
    ] + [const_spec(a) for a in wargs]

    out = pl.pallas_call(
        _body,
        grid=grid,
        in_specs=in_specs,
        out_specs=pl.BlockSpec((_G, Wf2.shape[1]), lambda i: (i, 0)),
        out_shape=jax.ShapeDtypeStruct((B, Wf2.shape[1]), jnp.float32),
    )(nodes, adj, *wargs)
    return out


# speedup vs baseline: 1.6822x; 1.5868x over previous
import functools

import jax
import jax.numpy as jnp
from jax.experimental import pallas as pl
from jax.experimental.pallas import tpu as pltpu

_NUM_LAYERS = 7
_HALF = 4
_N = 128
_G = 32

_dot = functools.partial(jax.lax.dot_general,
                         preferred_element_type=jnp.float32)


def _nn(a, b):
    return _dot(a, b, (((1,), (0,)), ((), ())))


def _nt(a, b):
    return _dot(a, b, (((1,), (1,)), ((), ())))


def _body(nodes_ref, adj_ref, *refs):
    out_ref = refs[-1]
    wrefs = [r[...] for r in refs[:-1]]

    idx = 0
    layers = []
    for i in range(_NUM_LAYERS):
        n = 3 if i < _HALF else 5
        layers.append(tuple(wrefs[idx:idx + n]))
        idx += n
    Wf1a, Wf1b, Wf1c, Wf1d, bf1, Wf2, bf2 = wrefs[idx:]

    ones_row = jnp.ones((1, _N), jnp.float32)
    invdegs = []
    adjs = []
    for g in range(_G):
        a = adj_ref[g]
        adjs.append(a)
        deg = _nt(ones_row, a)
        invdegs.append(1.0 / jnp.maximum(deg, 1e-6))

    nodes = nodes_ref[...].reshape(_G * _N, nodes_ref.shape[2])

    hT = None
    outs = []
    for i in range(_NUM_LAYERS):
        if i < _HALF:
            WtT, WbT, bT = layers[i]
            if i == 0:
                qT = _nt(WtT, nodes)
                pT = _nt(WbT, nodes)
            else:
                qT = _nn(WtT, hT)
                pT = _nn(WbT, hT)
        else:
            WthT, WtsT, WbhT, WbsT, bT = layers[i]
            sT = outs[_NUM_LAYERS - 1 - i]
            qT = _nn(WthT, hT) + _nn(WtsT, sT)
            pT = _nn(WbhT, hT) + _nn(WbsT, sT)
        nbT = jnp.concatenate(
            [_nt(pT[:, g * _N:(g + 1) * _N], adjs[g]) * invdegs[g]
             for g in range(_G)], axis=1)
        hT = jax.nn.relu(qT + nbT + bT)
        outs.append(hT)

    mxs, mns, sms = [], [], []
    for g in range(_G):
        hg = hT[:, g * _N:(g + 1) * _N]
        mxs.append(jnp.max(hg, axis=1, keepdims=True))
        mns.append(jnp.min(hg, axis=1, keepdims=True))
        sms.append(jnp.sum(hg, axis=1, keepdims=True))
    MXT = jnp.concatenate(mxs, axis=1)
    MNT = jnp.concatenate(mns, axis=1)
    SMT = jnp.concatenate(sms, axis=1)
    AVT = SMT * (1.0 / _N)

    r = jax.lax.broadcasted_iota(jnp.int32, (_G, _G), 0)
    c = jax.lax.broadcasted_iota(jnp.int32, (_G, _G), 1)
    eye = (r == c).astype(jnp.float32)
    MX = _nt(eye, MXT)
    MN = _nt(eye, MNT)
    AV = _nt(eye, AVT)
    SM = _nt(eye, SMT)

    hid = jax.nn.relu(
        _nn(MX, Wf1a) + _nn(MN, Wf1b) + _nn(AV, Wf1c) + _nn(SM, Wf1d) + bf1)
    out_ref[...] = _nn(hid, Wf2) + bf2


def kernel(nodes, adj, W0, W1, W2, W3, W4, W5, W6,
           b0, b1, b2, b3, b4, b5, b6, Wf1, bf1, Wf2, bf2):
    B, N, F0 = nodes.shape
    Ws = (W0, W1, W2, W3, W4, W5, W6)
    bs = (b0, b1, b2, b3, b4, b5, b6)

    feat_dims = [F0] + [W.shape[1] for W in Ws]
    wargs = []
    for i in range(_NUM_LAYERS):
        d = Ws[i].shape[0] // 2
        Wt, Wb = Ws[i][:d], Ws[i][d:]
        if i < _HALF:
            wargs += [Wt.T, Wb.T]
        else:
            hd = feat_dims[i]
            wargs += [Wt[:hd].T, Wt[hd:].T, Wb[:hd].T, Wb[hd:].T]
        wargs.append(bs[i].reshape(-1, 1))
    k = Wf1.shape[0] // 4
    wargs += [Wf1[:k], Wf1[k:2 * k], Wf1[2 * k:3 * k], Wf1[3 * k:],
              bf1.reshape(1, -1), Wf2, bf2.reshape(1, -1)]

    grid = (B // _G,)
    const_spec = lambda a: pl.BlockSpec(a.shape, lambda i: (0,) * a.ndim)
    in_specs = [
        pl.BlockSpec((_G, N, F0), lambda i: (i, 0, 0)),
        pl.BlockSpec((_G, N, N), lambda i: (i, 0, 0)),
    ] + [const_spec(a) for a in wargs]

    out = pl.pallas_call(
        _body,
        grid=grid,
        in_specs=in_specs,
        out_specs=pl.BlockSpec((_G, Wf2.shape[1]), lambda i: (i, 0)),
        out_shape=jax.ShapeDtypeStruct((B, Wf2.shape[1]), jnp.float32),
        compiler_params=pltpu.CompilerParams(
            dimension_semantics=("parallel",)),
    )(nodes, adj, *wargs)
    return out
